# Initial kernel scaffold; baseline (speedup 1.0000x reference)
import jax
import jax.numpy as jnp
from jax.experimental import pallas as pl


def kernel(points, cell_points):
    raise NotImplementedError("write your pallas kernel here")


# speedup vs baseline: 6.8004x; 6.8004x over previous
import functools

import jax
import jax.numpy as jnp
from jax.experimental import pallas as pl
from jax.experimental.pallas import tpu as pltpu

_Q = 8192
_N = 16384
_K = 11
_BQ = 128


def _voronoi_block(points_ref, ct_ref, out_ref, d2_ref):
    p = points_ref[...]
    ct = ct_ref[...]
    p2 = jnp.sum(p * p, axis=1, keepdims=True)
    c2 = jnp.sum(ct * ct, axis=0, keepdims=True)
    mm = jax.lax.dot_general(
        p, ct, (((1,), (0,)), ((), ())),
        preferred_element_type=jnp.float32)
    d2_ref[...] = (p2 + c2) - 2.0 * mm

    def select_min():
        d2cur = d2_ref[...]
        lane_iota = jax.lax.broadcasted_iota(jnp.int32, (_BQ, _N), 1)
        m = jnp.min(d2cur, axis=1, keepdims=True)
        idx = jnp.min(jnp.where(d2cur == m, lane_iota, _N),
                      axis=1, keepdims=True)
        onehot = (lane_iota == idx)
        d2_ref[...] = jnp.where(onehot, jnp.float32(jnp.inf), d2cur)
        coords = jax.lax.dot_general(
            onehot.astype(jnp.float32), ct, (((1,), (1,)), ((), ())),
            preferred_element_type=jnp.float32,
            precision=jax.lax.Precision.HIGHEST)
        return coords

    c0 = select_min()
    pc0 = p - c0

    def body(_, running):
        ck = select_min()
        e = ck - c0
        el = jnp.sqrt(jnp.sum(e * e, axis=1, keepdims=True))
        vl = jnp.sum(pc0 * e, axis=1, keepdims=True) / el
        sq = (vl - el / 2.0) ** 2
        return jnp.minimum(running, sq)

    running = jnp.full((_BQ, 1), jnp.inf, dtype=jnp.float32)
    out_ref[...] = jax.lax.fori_loop(0, _K - 1, body, running)


@jax.jit
def kernel(points, cell_points):
    ct = cell_points.T
    grid = _Q // _BQ
    out = pl.pallas_call(
        _voronoi_block,
        grid=(grid,),
        in_specs=[
            pl.BlockSpec((_BQ, 3), lambda i: (i, 0)),
            pl.BlockSpec((3, _N), lambda i: (0, 0)),
        ],
        out_specs=pl.BlockSpec((_BQ, 1), lambda i: (i, 0)),
        out_shape=jax.ShapeDtypeStruct((_Q, 1), jnp.float32),
        scratch_shapes=[pltpu.VMEM((_BQ, _N), jnp.float32)],
        compiler_params=pltpu.CompilerParams(
            dimension_semantics=("arbitrary",),
        ),
    )(points, ct)
    return out.reshape(_Q)


# speedup vs baseline: 13.5475x; 1.9922x over previous
import jax
import jax.numpy as jnp
from jax.experimental import pallas as pl
from jax.experimental.pallas import tpu as pltpu

_Q = 8192
_N = 16384
_K = 11
_BQ = 128


def _voronoi_block(points_ref, ct_ref, out_ref, d2_ref, e_ref):
    p = points_ref[...]
    ct = ct_ref[...]
    p2 = jnp.sum(p * p, axis=1, keepdims=True)
    c2 = jnp.sum(ct * ct, axis=0, keepdims=True)
    mm = jax.lax.dot_general(
        p, ct, (((1,), (0,)), ((), ())),
        preferred_element_type=jnp.float32)
    d2 = (p2 + c2) - 2.0 * mm
    d2_ref[...] = d2
    m0 = jnp.min(d2, axis=1, keepdims=True)

    lane_iota = jax.lax.broadcasted_iota(jnp.int32, (_BQ, _N), 1)
    idx0 = jnp.min(jnp.where(d2 == m0, lane_iota, _N),
                   axis=1, keepdims=True)
    onehot0 = (lane_iota == idx0)
    c0 = jax.lax.dot_general(
        onehot0.astype(jnp.float32), ct, (((1,), (1,)), ((), ())),
        preferred_element_type=jnp.float32,
        precision=jax.lax.Precision.HIGHEST)

    cc = jax.lax.dot_general(
        c0, ct, (((1,), (0,)), ((), ())),
        preferred_element_type=jnp.float32,
        precision=jax.lax.Precision.HIGHEST)
    c02 = jnp.sum(c0 * c0, axis=1, keepdims=True)
    e_ref[...] = (c02 + c2) - 2.0 * cc

    d2m = jnp.where(onehot0, jnp.float32(jnp.inf), d2)
    d2_ref[...] = d2m
    m1 = jnp.min(d2m, axis=1, keepdims=True)

    def body(_, carry):
        m, acc = carry
        d2cur = d2_ref[...]
        epan = e_ref[...]
        is_min = d2cur == m
        d2nxt = jnp.where(is_min, jnp.float32(jnp.inf), d2cur)
        d2_ref[...] = d2nxt
        mnxt = jnp.min(d2nxt, axis=1, keepdims=True)
        ek = jnp.max(jnp.where(is_min, epan, -jnp.float32(jnp.inf)),
                     axis=1, keepdims=True)
        sq = (m - m0) ** 2 / (4.0 * ek)
        return mnxt, jnp.minimum(acc, sq)

    acc0 = jnp.full((_BQ, 1), jnp.inf, dtype=jnp.float32)
    _, acc = jax.lax.fori_loop(0, _K - 1, body, (m1, acc0))
    out_ref[...] = acc


@jax.jit
def kernel(points, cell_points):
    ct = cell_points.T
    grid = _Q // _BQ
    out = pl.pallas_call(
        _voronoi_block,
        grid=(grid,),
        in_specs=[
            pl.BlockSpec((_BQ, 3), lambda i: (i, 0)),
            pl.BlockSpec((3, _N), lambda i: (0, 0)),
        ],
        out_specs=pl.BlockSpec((_BQ, 1), lambda i: (i, 0)),
        out_shape=jax.ShapeDtypeStruct((_Q, 1), jnp.float32),
        scratch_shapes=[
            pltpu.VMEM((_BQ, _N), jnp.float32),
            pltpu.VMEM((_BQ, _N), jnp.float32),
        ],
        compiler_params=pltpu.CompilerParams(
            dimension_semantics=("arbitrary",),
        ),
    )(points, ct)
    return out.reshape(_Q)


# speedup vs baseline: 13.7250x; 1.0131x over previous
import jax
import jax.numpy as jnp
from jax.experimental import pallas as pl
from jax.experimental.pallas import tpu as pltpu

_Q = 8192
_N = 16384
_K = 11
_BQ = 128


def _voronoi_block(points_ref, ct_ref, out_ref, d2_ref, a2_ref, e_ref):
    p = points_ref[...]
    ct = ct_ref[...]
    p2 = jnp.sum(p * p, axis=1, keepdims=True)
    c2 = jnp.sum(ct * ct, axis=0, keepdims=True)
    mm = jax.lax.dot_general(
        p, ct, (((1,), (0,)), ((), ())),
        preferred_element_type=jnp.float32)
    d2 = (p2 + c2) - 2.0 * mm
    m0 = jnp.min(d2, axis=1, keepdims=True)

    mmh = jax.lax.dot_general(
        p, ct, (((1,), (0,)), ((), ())),
        preferred_element_type=jnp.float32,
        precision=jax.lax.Precision.HIGHEST)
    a2 = (p2 + c2) - 2.0 * mmh
    a2_ref[...] = a2

    lane_iota = jax.lax.broadcasted_iota(jnp.int32, (_BQ, _N), 1)
    idx0 = jnp.min(jnp.where(d2 == m0, lane_iota, _N),
                   axis=1, keepdims=True)
    onehot = lane_iota == idx0
    neg = -jnp.float32(jnp.inf)

    def gather(panel):
        return jnp.max(jnp.where(onehot, panel, neg), axis=1, keepdims=True)

    m0a = gather(a2)
    c0x = gather(ct[0:1, :])
    c0y = gather(ct[1:2, :])
    c0z = gather(ct[2:3, :])
    c0 = jnp.concatenate([c0x, c0y, c0z], axis=1)

    cc = jax.lax.dot_general(
        c0, ct, (((1,), (0,)), ((), ())),
        preferred_element_type=jnp.float32,
        precision=jax.lax.Precision.HIGHEST)
    c02 = jnp.sum(c0 * c0, axis=1, keepdims=True)
    e_ref[...] = (c02 + c2) - 2.0 * cc

    d2m = jnp.where(onehot, jnp.float32(jnp.inf), d2)
    d2_ref[...] = d2m
    m1 = jnp.min(d2m, axis=1, keepdims=True)

    def body(_, carry):
        m, acc = carry
        d2cur = d2_ref[...]
        is_min = d2cur == m
        d2nxt = jnp.where(is_min, jnp.float32(jnp.inf), d2cur)
        d2_ref[...] = d2nxt
        mnxt = jnp.min(d2nxt, axis=1, keepdims=True)
        mka = jnp.max(jnp.where(is_min, a2_ref[...], neg),
                      axis=1, keepdims=True)
        ek = jnp.max(jnp.where(is_min, e_ref[...], neg),
                     axis=1, keepdims=True)
        sq = (mka - m0a) ** 2 / (4.0 * ek)
        return mnxt, jnp.minimum(acc, sq)

    acc0 = jnp.full((_BQ, 1), jnp.inf, dtype=jnp.float32)
    _, acc = jax.lax.fori_loop(0, _K - 1, body, (m1, acc0))
    out_ref[...] = acc


@jax.jit
def kernel(points, cell_points):
    ct = cell_points.T
    grid = _Q // _BQ
    out = pl.pallas_call(
        _voronoi_block,
        grid=(grid,),
        in_specs=[
            pl.BlockSpec((_BQ, 3), lambda i: (i, 0)),
            pl.BlockSpec((3, _N), lambda i: (0, 0)),
        ],
        out_specs=pl.BlockSpec((_BQ, 1), lambda i: (i, 0)),
        out_shape=jax.ShapeDtypeStruct((_Q, 1), jnp.float32),
        scratch_shapes=[
            pltpu.VMEM((_BQ, _N), jnp.float32),
            pltpu.VMEM((_BQ, _N), jnp.float32),
            pltpu.VMEM((_BQ, _N), jnp.float32),
        ],
        compiler_params=pltpu.CompilerParams(
            dimension_semantics=("arbitrary",),
        ),
    )(points, ct)
    return out.reshape(_Q)
